# baseline (speedup 1.0000x reference)
import functools
import math

import jax
import jax.numpy as jnp
from jax import lax
from jax.experimental import pallas as pl
from jax.experimental.pallas import tpu as pltpu
from jax.experimental.pallas import tpu_sc as plsc

D_MODEL = 64
SCALE = math.sqrt(D_MODEL)

NC = 2
NS = 16
NW = NC * NS
LANES = 16

CHUNK = 512


def _make_emb_kernel(B: int):
  assert B % (NW * CHUNK) == 0
  b_per_w = B // NW
  n_chunks = b_per_w // CHUNK

  mesh = plsc.VectorSubcoreMesh(core_axis_name="c", subcore_axis_name="s")

  @functools.partial(
      pl.kernel,
      out_type=jax.ShapeDtypeStruct((B, D_MODEL), jnp.float32),
      mesh=mesh,
      scratch_types=[
          pltpu.VMEM((CHUNK,), jnp.int32),
          pltpu.VMEM((CHUNK, D_MODEL), jnp.float32),
          pltpu.SemaphoreType.DMA,
      ],
      compiler_params=pltpu.CompilerParams(use_tc_tiling_on_sc=False),
  )
  def emb(idx_hbm, table_hbm, out_hbm, idx_v, rows_v, sem):
    wid = lax.axis_index("s") * NC + lax.axis_index("c")
    wbase = wid * b_per_w

    def chunk_body(g, carry):
      base = wbase + g * CHUNK
      pltpu.sync_copy(idx_hbm.at[pl.ds(base, CHUNK)], idx_v)
      pltpu.async_copy(table_hbm.at[idx_v], rows_v, sem).wait()

      @plsc.parallel_loop(0, CHUNK, 1, unroll=8)
      def _scale(i):
        for j in range(D_MODEL // LANES):
          sl = pl.ds(j * LANES, LANES)
          rows_v[i, sl] = rows_v[i, sl] * SCALE

      pltpu.sync_copy(rows_v, out_hbm.at[pl.ds(base, CHUNK)])
      return carry

    lax.fori_loop(0, n_chunks, chunk_body, 0)

  return emb


def kernel(x, lut):
  idx = x.reshape(-1).astype(jnp.int32)
  out = _make_emb_kernel(idx.shape[0])(idx, lut)
  return out.reshape(*x.shape, D_MODEL)


# speedup vs baseline: 1.0903x; 1.0903x over previous
import functools
import math

import jax
import jax.numpy as jnp
from jax import lax
from jax.experimental import pallas as pl
from jax.experimental.pallas import tpu as pltpu
from jax.experimental.pallas import tpu_sc as plsc

D_MODEL = 64
SCALE = math.sqrt(D_MODEL)

NC = 2
NS = 16
NW = NC * NS
LANES = 16

CHUNK = 512
NBUF = 3


def _make_emb_kernel(B: int):
  assert B % (NW * CHUNK) == 0
  b_per_w = B // NW
  n_chunks = b_per_w // CHUNK

  mesh = plsc.VectorSubcoreMesh(core_axis_name="c", subcore_axis_name="s")

  @functools.partial(
      pl.kernel,
      out_type=jax.ShapeDtypeStruct((B, D_MODEL), jnp.float32),
      mesh=mesh,
      scratch_types=(
          [pltpu.VMEM((CHUNK,), jnp.int32) for _ in range(NBUF)]
          + [pltpu.VMEM((CHUNK, D_MODEL), jnp.float32) for _ in range(NBUF)]
          + [
              pltpu.SemaphoreType.DMA((NBUF,)),
              pltpu.SemaphoreType.DMA((NBUF,)),
              pltpu.SemaphoreType.DMA((NBUF,)),
          ]
      ),
      compiler_params=pltpu.CompilerParams(use_tc_tiling_on_sc=False),
  )
  def emb(idx_hbm, table_hbm, out_hbm, ibuf0, ibuf1, ibuf2, rows0, rows1,
          rows2, isem, gsem, osem):
    ibufs = (ibuf0, ibuf1, ibuf2)
    rowss = (rows0, rows1, rows2)
    wid = lax.axis_index("s") * NC + lax.axis_index("c")
    wbase = wid * b_per_w

    def idx_slice(t):
      return idx_hbm.at[pl.ds(wbase + t * CHUNK, CHUNK)]

    def out_slice(t):
      return out_hbm.at[pl.ds(wbase + t * CHUNK, CHUNK)]

    def issue_idx(t, slot):
      pltpu.async_copy(idx_slice(t), ibufs[slot], isem.at[slot])

    def wait_idx(t, slot):
      pltpu.make_async_copy(idx_slice(t), ibufs[slot], isem.at[slot]).wait()

    def issue_gather(slot):
      pltpu.async_copy(table_hbm.at[ibufs[slot]], rowss[slot],
                       gsem.at[slot])

    def wait_gather(slot):
      pltpu.make_async_copy(table_hbm.at[ibufs[slot]], rowss[slot],
                            gsem.at[slot]).wait()

    def issue_out(t, slot):
      pltpu.async_copy(rowss[slot], out_slice(t), osem.at[slot])

    def wait_out(t, slot):
      pltpu.make_async_copy(rowss[slot], out_slice(t), osem.at[slot]).wait()

    issue_idx(0, 0)
    issue_idx(1, 1)
    wait_idx(0, 0)
    issue_gather(0)

    n_outer = (n_chunks + NBUF - 1) // NBUF

    def outer(ti, carry):
      for b in range(NBUF):
        t = ti * NBUF + b
        b1 = (b + 1) % NBUF
        b2 = (b + 2) % NBUF

        @pl.when(t + 2 < n_chunks)
        def _():
          issue_idx(t + 2, b2)

        @pl.when(t + 1 < n_chunks)
        def _():
          @pl.when(t >= 2)
          def _():
            wait_out(t - 2, b1)

          wait_idx(t + 1, b1)
          issue_gather(b1)

        @pl.when(t < n_chunks)
        def _():
          wait_gather(b)

          rv = rowss[b]

          @plsc.parallel_loop(0, CHUNK, 1, unroll=4)
          def _scale(i):
            for j in range(D_MODEL // LANES):
              sl = pl.ds(j * LANES, LANES)
              rv[i, sl] = rv[i, sl] * SCALE

          issue_out(t, b)

      return carry

    lax.fori_loop(0, n_outer, outer, 0)

    for t in range(n_chunks - NBUF, n_chunks):
      wait_out(t, t % NBUF)

  return emb


def kernel(x, lut):
  idx = x.reshape(-1).astype(jnp.int32)
  out = _make_emb_kernel(idx.shape[0])(idx, lut)
  return out.reshape(*x.shape, D_MODEL)
